# baseline (speedup 1.0000x reference)
import functools

import jax
import jax.numpy as jnp
from jax import lax
from jax.experimental import pallas as pl
from jax.experimental.pallas import tpu as pltpu
from jax.experimental.pallas import tpu_sc as plsc

N = 10000
E = 320000
D = 128
H = 64
C = 47

NC = 2
NS = 16
NW = NC * NS
ET = E // NW
K = 125
NCHUNK = ET // K
NBUF = 4

ROWS_A = N // NS - (N // NS) % 8
ROWS_TAIL = N - ROWS_A * NS
ROWS_B = ROWS_A // 2


def _mesh():
    return plsc.VectorSubcoreMesh(core_axis_name="c", subcore_axis_name="s")


def _deg_body(dst_hbm, zeros_hbm, out_hbm, didx, ones_v, zb, acc, sem):
    c = lax.axis_index("c")
    s = lax.axis_index("s")
    w = s * NC + c
    pltpu.sync_copy(dst_hbm.at[pl.ds(w * NCHUNK, NCHUNK)], didx)
    def fill(i, _):
        ones_v[pl.ds(i * 16, 16)] = jnp.full((16,), 1.0, jnp.float32)
        return _
    lax.fori_loop(0, 8, fill, 0)
    pltpu.sync_copy(zeros_hbm, zb)
    pltpu.sync_copy(zb.at[pl.ds(0, ROWS_A)], acc.at[pl.ds(s * ROWS_A, ROWS_A)])
    @pl.when(s == NS - 1)
    def _():
        pltpu.sync_copy(zb.at[pl.ds(0, ROWS_TAIL)],
                        acc.at[pl.ds(NS * ROWS_A, ROWS_TAIL)])
    plsc.subcore_barrier()
    def body(j, carry):
        pltpu.async_copy(ones_v.at[pl.ds(0, K)], acc.at[didx.at[j]], sem,
                         add=True)
        @pl.when(j >= 8)
        def _w():
            pltpu.make_async_copy(ones_v.at[pl.ds(0, K)],
                                  acc.at[didx.at[j]], sem).wait()
        return carry
    lax.fori_loop(0, NCHUNK, body, 0)
    def drain(j, _):
        pltpu.make_async_copy(ones_v.at[pl.ds(0, K)],
                              acc.at[didx.at[j]], sem).wait()
        return _
    lax.fori_loop(0, 8, drain, 0)
    plsc.subcore_barrier()
    pltpu.sync_copy(acc.at[pl.ds(s * ROWS_A, ROWS_A)], zb.at[pl.ds(0, ROWS_A)])
    pltpu.sync_copy(zb.at[pl.ds(0, ROWS_A)],
                    out_hbm.at[pl.ds(c * N + s * ROWS_A, ROWS_A)])
    @pl.when(s == NS - 1)
    def _():
        pltpu.sync_copy(acc.at[pl.ds(NS * ROWS_A, ROWS_TAIL)],
                        zb.at[pl.ds(0, ROWS_TAIL)])
        pltpu.sync_copy(zb.at[pl.ds(0, ROWS_TAIL)],
                        out_hbm.at[pl.ds(c * N + NS * ROWS_A, ROWS_TAIL)])


def _sc_degree(dst, zeros1):
    return pl.kernel(
        _deg_body,
        out_type=jax.ShapeDtypeStruct((NC * N,), jnp.float32),
        mesh=_mesh(),
        compiler_params=pltpu.CompilerParams(use_tc_tiling_on_sc=False),
        scratch_types=[
            pltpu.VMEM((NCHUNK, K), jnp.int32),
            pltpu.VMEM((128,), jnp.float32),
            pltpu.VMEM((ROWS_A,), jnp.float32),
            pltpu.VMEM_SHARED((N,), jnp.float32),
            pltpu.SemaphoreType.DMA,
        ],
    )(dst, zeros1)


def _prop_body(zt_hbm, src_hbm, dst_hbm, zeros_hbm, out_hbm,
               sidx, didx, rows0, rows1, rows2, rows3, zb, acc,
               gs0, gs1, gs2, gs3, ss0, ss1, ss2, ss3):
    rows = (rows0, rows1, rows2, rows3)
    gsems = (gs0, gs1, gs2, gs3)
    ssems = (ss0, ss1, ss2, ss3)
    c = lax.axis_index("c")
    s = lax.axis_index("s")
    w = s * NC + c
    pltpu.sync_copy(src_hbm.at[pl.ds(w * NCHUNK, NCHUNK)], sidx)
    pltpu.sync_copy(dst_hbm.at[pl.ds(w * NCHUNK, NCHUNK)], didx)
    pltpu.sync_copy(zeros_hbm, zb)
    pltpu.sync_copy(zb, acc.at[pl.ds(s * ROWS_A, ROWS_B)])
    pltpu.sync_copy(zb, acc.at[pl.ds(s * ROWS_A + ROWS_B, ROWS_B)])
    @pl.when(s == NS - 1)
    def _():
        pltpu.sync_copy(zb.at[pl.ds(0, ROWS_TAIL)],
                        acc.at[pl.ds(NS * ROWS_A, ROWS_TAIL)])
    plsc.subcore_barrier()

    def gather(j, b):
        return pltpu.make_async_copy(zt_hbm.at[sidx.at[j]], rows[b],
                                     gsems[b])

    def scat_start(j, b):
        pltpu.async_copy(rows[b], acc.at[didx.at[j]], ssems[b], add=True)

    def scat_wait(j, b):
        pltpu.make_async_copy(rows[b], acc.at[didx.at[j]], ssems[b]).wait()

    gather(0, 0).start()
    gather(1, 1).start()
    def body(i, carry):
        for b in range(NBUF):
            j = NBUF * i + b
            b2 = (b + 2) % NBUF
            gather(j, b).wait()
            scat_start(j, b)
            @pl.when(j + 2 < NCHUNK)
            def _pre(b2=b2, j=j):
                @pl.when(j >= 2)
                def _w():
                    scat_wait(j - 2, b2)
                gather(j + 2, b2).start()
        return carry
    lax.fori_loop(0, NCHUNK // NBUF, body, 0)
    for b in range(NBUF):
        scat_wait(NCHUNK - NBUF + b, b)
    plsc.subcore_barrier()
    pltpu.sync_copy(acc.at[pl.ds(s * ROWS_A, ROWS_B)], zb)
    pltpu.sync_copy(zb, out_hbm.at[c, pl.ds(s * ROWS_A, ROWS_B)])
    pltpu.sync_copy(acc.at[pl.ds(s * ROWS_A + ROWS_B, ROWS_B)], zb)
    pltpu.sync_copy(zb, out_hbm.at[c, pl.ds(s * ROWS_A + ROWS_B, ROWS_B)])
    @pl.when(s == NS - 1)
    def _():
        pltpu.sync_copy(acc.at[pl.ds(NS * ROWS_A, ROWS_TAIL)],
                        zb.at[pl.ds(0, ROWS_TAIL)])
        pltpu.sync_copy(zb.at[pl.ds(0, ROWS_TAIL)],
                        out_hbm.at[c, pl.ds(NS * ROWS_A, ROWS_TAIL)])


def _sc_propagate(zt, src, dst, zeros2):
    return pl.kernel(
        _prop_body,
        out_type=jax.ShapeDtypeStruct((NC, N, H), jnp.float32),
        mesh=_mesh(),
        compiler_params=pltpu.CompilerParams(use_tc_tiling_on_sc=False),
        scratch_types=(
            [pltpu.VMEM((NCHUNK, K), jnp.int32)] * 2
            + [pltpu.VMEM((K, H), jnp.float32)] * NBUF
            + [pltpu.VMEM((ROWS_B, H), jnp.float32),
               pltpu.VMEM_SHARED((N, H), jnp.float32)]
            + [pltpu.SemaphoreType.DMA] * (2 * NBUF)
        ),
    )(zt, src, dst, zeros2)


BLK = 1000
GRID = N // BLK


def _tcb_body(x_ref, d0_ref, d1_ref, m0_ref, m1a_ref, v0_ref,
              zt0_ref, z1a_ref, dinv_ref):
    xb = x_ref[...]
    dv = lax.rsqrt(1.0 + d0_ref[...] + d1_ref[...])
    dinv_ref[...] = dv
    zt0_ref[...] = dv * (jnp.dot(xb, m0_ref[...],
                                 preferred_element_type=jnp.float32)
                         + v0_ref[...])
    z1a_ref[...] = jnp.dot(xb, m1a_ref[...],
                           preferred_element_type=jnp.float32)


def _tc_b(x, d0, d1, m0, m1a, v0):
    row = pl.BlockSpec((BLK, H), lambda i: (i, 0))
    col = pl.BlockSpec((BLK, 1), lambda i: (i, 0))
    return pl.pallas_call(
        _tcb_body,
        grid=(GRID,),
        in_specs=[
            pl.BlockSpec((BLK, D), lambda i: (i, 0)),
            col,
            col,
            pl.BlockSpec((D, H), lambda i: (0, 0)),
            pl.BlockSpec((D, H), lambda i: (0, 0)),
            pl.BlockSpec((1, H), lambda i: (0, 0)),
        ],
        out_specs=[row, row, col],
        out_shape=[jax.ShapeDtypeStruct((N, H), jnp.float32),
                   jax.ShapeDtypeStruct((N, H), jnp.float32),
                   jax.ShapeDtypeStruct((N, 1), jnp.float32)],
    )(x, d0, d1, m0, m1a, v0)


def _tcc_body(p_ref, zt0_ref, z1a_ref, dinv_ref, m1b_ref, v1_ref, zt1_ref):
    dv = dinv_ref[...]
    u0 = dv * (p_ref[0] + p_ref[1] + zt0_ref[...])
    z1 = z1a_ref[...] + jnp.dot(u0, m1b_ref[...],
                                preferred_element_type=jnp.float32) + v1_ref[...]
    zt1_ref[...] = dv * z1


def _tc_c(p, zt0, z1a, dinv, m1b, v1):
    row = pl.BlockSpec((BLK, H), lambda i: (i, 0))
    return pl.pallas_call(
        _tcc_body,
        grid=(GRID,),
        in_specs=[
            pl.BlockSpec((NC, BLK, H), lambda i: (0, i, 0)),
            row,
            row,
            pl.BlockSpec((BLK, 1), lambda i: (i, 0)),
            pl.BlockSpec((H, H), lambda i: (0, 0)),
            pl.BlockSpec((1, H), lambda i: (0, 0)),
        ],
        out_specs=row,
        out_shape=jax.ShapeDtypeStruct((N, H), jnp.float32),
    )(p, zt0, z1a, dinv, m1b, v1)


def _tcd_body(p_ref, zt1_ref, dinv_ref, cls_ref, v2_ref, out_ref):
    u1 = dinv_ref[...] * (p_ref[0] + p_ref[1] + zt1_ref[...])
    out_ref[...] = jnp.dot(u1, cls_ref[...],
                           preferred_element_type=jnp.float32) + v2_ref[...]


def _tc_d(p, zt1, dinv, cls_W, v2):
    return pl.pallas_call(
        _tcd_body,
        grid=(GRID,),
        in_specs=[
            pl.BlockSpec((NC, BLK, H), lambda i: (0, i, 0)),
            pl.BlockSpec((BLK, H), lambda i: (i, 0)),
            pl.BlockSpec((BLK, 1), lambda i: (i, 0)),
            pl.BlockSpec((H, C), lambda i: (0, 0)),
            pl.BlockSpec((1, C), lambda i: (0, 0)),
        ],
        out_specs=pl.BlockSpec((BLK, C), lambda i: (i, 0)),
        out_shape=jax.ShapeDtypeStruct((N, C), jnp.float32),
    )(p, zt1, dinv, cls_W, v2)


@jax.jit
def kernel(x, edge_index,
           c0_p0_W, c0_p0_b, c0_p1_W, c0_p1_b, c0_g0_W, c0_g0_b,
           c0_g1_W, c0_g1_b,
           c1_p0_W, c1_p0_b, c1_p1_W, c1_p1_b, c1_g0_W, c1_g0_b,
           c1_g1_W, c1_g1_b,
           cls_W, cls_b):
    src = edge_index[0]
    dst = edge_index[1]

    m0 = 2.0 * (c0_p0_W @ c0_g0_W) + c0_p1_W @ c0_g1_W
    v0 = 2.0 * (c0_p0_b @ c0_g0_W) + c0_p1_b @ c0_g1_W
    cb0 = 2.0 * c0_g0_b + c0_g1_b
    m1a = 2.0 * (c1_p0_W @ c1_g0_W)
    m1b = c1_p1_W @ c1_g1_W
    v1 = 2.0 * (c1_p0_b @ c1_g0_W) + c1_p1_b @ c1_g1_W + cb0 @ m1b
    cb1 = 2.0 * c1_g0_b + c1_g1_b
    v2 = cb1 @ cls_W + cls_b

    zeros1 = jnp.zeros((ROWS_A,), jnp.float32)
    zeros2 = jnp.zeros((ROWS_B, H), jnp.float32)

    src2 = src.reshape(NW * NCHUNK, K)
    dst2 = dst.reshape(NW * NCHUNK, K)
    deg_parts = _sc_degree(dst2, zeros1)
    d0 = deg_parts[:N].reshape(N, 1)
    d1 = deg_parts[N:].reshape(N, 1)
    zt0, z1a, dinv = _tc_b(x, d0, d1, m0, m1a, v0.reshape(1, H))
    p0 = _sc_propagate(zt0, src2, dst2, zeros2)
    zt1 = _tc_c(p0, zt0, z1a, dinv, m1b, v1.reshape(1, H))
    p1 = _sc_propagate(zt1, src2, dst2, zeros2)
    return _tc_d(p1, zt1, dinv, cls_W, v2.reshape(1, C))


# speedup vs baseline: 1.0333x; 1.0333x over previous
import functools

import jax
import jax.numpy as jnp
from jax import lax
from jax.experimental import pallas as pl
from jax.experimental.pallas import tpu as pltpu
from jax.experimental.pallas import tpu_sc as plsc

N = 10000
E = 320000
D = 128
H = 64
C = 47

NC = 2
NS = 16
NW = NC * NS
ET = E // NW
K = 125
NCHUNK = ET // K
NBUF = 4

ROWS_A = N // NS - (N // NS) % 8
ROWS_TAIL = N - ROWS_A * NS
ROWS_B = ROWS_A // 2


def _mesh():
    return plsc.VectorSubcoreMesh(core_axis_name="c", subcore_axis_name="s")


def _deg_body(dst_hbm, zeros_hbm, out_hbm, didx, ones_v, zb, acc, sem):
    c = lax.axis_index("c")
    s = lax.axis_index("s")
    w = s * NC + c
    pltpu.sync_copy(dst_hbm.at[pl.ds(w * NCHUNK, NCHUNK)], didx)
    def fill(i, _):
        ones_v[pl.ds(i * 16, 16)] = jnp.full((16,), 1.0, jnp.float32)
        return _
    lax.fori_loop(0, 8, fill, 0)
    pltpu.sync_copy(zeros_hbm, zb)
    pltpu.sync_copy(zb.at[pl.ds(0, ROWS_A)], acc.at[pl.ds(s * ROWS_A, ROWS_A)])
    @pl.when(s == NS - 1)
    def _():
        pltpu.sync_copy(zb.at[pl.ds(0, ROWS_TAIL)],
                        acc.at[pl.ds(NS * ROWS_A, ROWS_TAIL)])
    plsc.subcore_barrier()
    def body(j, carry):
        pltpu.async_copy(ones_v.at[pl.ds(0, K)], acc.at[didx.at[j]], sem,
                         add=True)
        @pl.when(j >= 8)
        def _w():
            pltpu.make_async_copy(ones_v.at[pl.ds(0, K)],
                                  acc.at[didx.at[j]], sem).wait()
        return carry
    lax.fori_loop(0, NCHUNK, body, 0)
    def drain(j, _):
        pltpu.make_async_copy(ones_v.at[pl.ds(0, K)],
                              acc.at[didx.at[j]], sem).wait()
        return _
    lax.fori_loop(0, 8, drain, 0)
    plsc.subcore_barrier()
    pltpu.sync_copy(acc.at[pl.ds(s * ROWS_A, ROWS_A)], zb.at[pl.ds(0, ROWS_A)])
    pltpu.sync_copy(zb.at[pl.ds(0, ROWS_A)],
                    out_hbm.at[pl.ds(c * N + s * ROWS_A, ROWS_A)])
    @pl.when(s == NS - 1)
    def _():
        pltpu.sync_copy(acc.at[pl.ds(NS * ROWS_A, ROWS_TAIL)],
                        zb.at[pl.ds(0, ROWS_TAIL)])
        pltpu.sync_copy(zb.at[pl.ds(0, ROWS_TAIL)],
                        out_hbm.at[pl.ds(c * N + NS * ROWS_A, ROWS_TAIL)])


def _sc_degree(dst, zeros1):
    return pl.kernel(
        _deg_body,
        out_type=jax.ShapeDtypeStruct((NC * N,), jnp.float32),
        mesh=_mesh(),
        compiler_params=pltpu.CompilerParams(use_tc_tiling_on_sc=False),
        scratch_types=[
            pltpu.VMEM((NCHUNK, K), jnp.int32),
            pltpu.VMEM((128,), jnp.float32),
            pltpu.VMEM((ROWS_A,), jnp.float32),
            pltpu.VMEM_SHARED((N,), jnp.float32),
            pltpu.SemaphoreType.DMA,
        ],
    )(dst, zeros1)


def _prop_body(zt_hbm, src_hbm, dst_hbm, zeros_hbm, out_hbm,
               sidx, didx, rows0, rows1, rows2, rows3, zb, acc,
               gs0, gs1, gs2, gs3, ss0, ss1, ss2, ss3):
    rows = (rows0, rows1, rows2, rows3)
    gsems = (gs0, gs1, gs2, gs3)
    ssems = (ss0, ss1, ss2, ss3)
    c = lax.axis_index("c")
    s = lax.axis_index("s")
    w = s * NC + c
    pltpu.sync_copy(src_hbm.at[pl.ds(w * NCHUNK, NCHUNK)], sidx)
    pltpu.sync_copy(dst_hbm.at[pl.ds(w * NCHUNK, NCHUNK)], didx)
    pltpu.sync_copy(zeros_hbm, zb)
    pltpu.sync_copy(zb, acc.at[pl.ds(s * ROWS_A, ROWS_B)])
    pltpu.sync_copy(zb, acc.at[pl.ds(s * ROWS_A + ROWS_B, ROWS_B)])
    @pl.when(s == NS - 1)
    def _():
        pltpu.sync_copy(zb.at[pl.ds(0, ROWS_TAIL)],
                        acc.at[pl.ds(NS * ROWS_A, ROWS_TAIL)])
    plsc.subcore_barrier()

    def gather(j, b):
        return pltpu.make_async_copy(zt_hbm.at[sidx.at[j]], rows[b],
                                     gsems[b])

    def scat_start(j, b):
        pltpu.async_copy(rows[b], acc.at[didx.at[j]], ssems[b], add=True)

    def scat_wait(j, b):
        pltpu.make_async_copy(rows[b], acc.at[didx.at[j]], ssems[b]).wait()

    gather(0, 0).start()
    gather(1, 1).start()
    def body(i, carry):
        for b in range(NBUF):
            j = NBUF * i + b
            b2 = (b + 2) % NBUF
            gather(j, b).wait()
            scat_start(j, b)
            @pl.when(j + 2 < NCHUNK)
            def _pre(b2=b2, j=j):
                @pl.when(j >= 2)
                def _w():
                    scat_wait(j - 2, b2)
                gather(j + 2, b2).start()
        return carry
    lax.fori_loop(0, NCHUNK // NBUF, body, 0)
    for b in range(NBUF):
        scat_wait(NCHUNK - NBUF + b, b)
    plsc.subcore_barrier()
    pltpu.sync_copy(acc.at[pl.ds(s * ROWS_A, ROWS_B)], zb)
    pltpu.sync_copy(zb, out_hbm.at[c, pl.ds(s * ROWS_A, ROWS_B)])
    pltpu.sync_copy(acc.at[pl.ds(s * ROWS_A + ROWS_B, ROWS_B)], zb)
    pltpu.sync_copy(zb, out_hbm.at[c, pl.ds(s * ROWS_A + ROWS_B, ROWS_B)])
    @pl.when(s == NS - 1)
    def _():
        pltpu.sync_copy(acc.at[pl.ds(NS * ROWS_A, ROWS_TAIL)],
                        zb.at[pl.ds(0, ROWS_TAIL)])
        pltpu.sync_copy(zb.at[pl.ds(0, ROWS_TAIL)],
                        out_hbm.at[c, pl.ds(NS * ROWS_A, ROWS_TAIL)])


def _sc_propagate(zt, src, dst, zeros2):
    return pl.kernel(
        _prop_body,
        out_type=jax.ShapeDtypeStruct((NC, N, H), jnp.float32),
        mesh=_mesh(),
        compiler_params=pltpu.CompilerParams(use_tc_tiling_on_sc=False),
        scratch_types=(
            [pltpu.VMEM((NCHUNK, K), jnp.int32)] * 2
            + [pltpu.VMEM((K, H), jnp.float32)] * NBUF
            + [pltpu.VMEM((ROWS_B, H), jnp.float32),
               pltpu.VMEM_SHARED((N, H), jnp.float32)]
            + [pltpu.SemaphoreType.DMA] * (2 * NBUF)
        ),
    )(zt, src, dst, zeros2)


BLK = 1000
GRID = N // BLK


def _tcb_body(x_ref, dinv_ref, m0_ref, m1a_ref, v0_ref, zt0_ref, z1a_ref):
    xb = x_ref[...]
    dv = dinv_ref[...]
    zt0_ref[...] = dv * (jnp.dot(xb, m0_ref[...],
                                 preferred_element_type=jnp.float32)
                         + v0_ref[...])
    z1a_ref[...] = jnp.dot(xb, m1a_ref[...],
                           preferred_element_type=jnp.float32)


def _tc_b(x, dinv, m0, m1a, v0):
    row = pl.BlockSpec((BLK, H), lambda i: (i, 0))
    return pl.pallas_call(
        _tcb_body,
        grid=(GRID,),
        in_specs=[
            pl.BlockSpec((BLK, D), lambda i: (i, 0)),
            pl.BlockSpec((BLK, 1), lambda i: (i, 0)),
            pl.BlockSpec((D, H), lambda i: (0, 0)),
            pl.BlockSpec((D, H), lambda i: (0, 0)),
            pl.BlockSpec((1, H), lambda i: (0, 0)),
        ],
        out_specs=[row, row],
        out_shape=[jax.ShapeDtypeStruct((N, H), jnp.float32),
                   jax.ShapeDtypeStruct((N, H), jnp.float32)],
    )(x, dinv, m0, m1a, v0)


def _tcc_body(p_ref, zt0_ref, z1a_ref, dinv_ref, m1b_ref, v1_ref, zt1_ref):
    dv = dinv_ref[...]
    u0 = dv * (p_ref[0] + p_ref[1] + zt0_ref[...])
    z1 = z1a_ref[...] + jnp.dot(u0, m1b_ref[...],
                                preferred_element_type=jnp.float32) + v1_ref[...]
    zt1_ref[...] = dv * z1


def _tc_c(p, zt0, z1a, dinv, m1b, v1):
    row = pl.BlockSpec((BLK, H), lambda i: (i, 0))
    return pl.pallas_call(
        _tcc_body,
        grid=(GRID,),
        in_specs=[
            pl.BlockSpec((NC, BLK, H), lambda i: (0, i, 0)),
            row,
            row,
            pl.BlockSpec((BLK, 1), lambda i: (i, 0)),
            pl.BlockSpec((H, H), lambda i: (0, 0)),
            pl.BlockSpec((1, H), lambda i: (0, 0)),
        ],
        out_specs=row,
        out_shape=jax.ShapeDtypeStruct((N, H), jnp.float32),
    )(p, zt0, z1a, dinv, m1b, v1)


def _tcd_body(p_ref, zt1_ref, dinv_ref, cls_ref, v2_ref, out_ref):
    u1 = dinv_ref[...] * (p_ref[0] + p_ref[1] + zt1_ref[...])
    out_ref[...] = jnp.dot(u1, cls_ref[...],
                           preferred_element_type=jnp.float32) + v2_ref[...]


def _tc_d(p, zt1, dinv, cls_W, v2):
    row = pl.BlockSpec((BLK, H), lambda i: (i, 0))
    return pl.pallas_call(
        _tcd_body,
        grid=(GRID,),
        in_specs=[
            pl.BlockSpec((NC, BLK, H), lambda i: (0, i, 0)),
            row,
            pl.BlockSpec((BLK, 1), lambda i: (i, 0)),
            pl.BlockSpec((H, C), lambda i: (0, 0)),
            pl.BlockSpec((1, C), lambda i: (0, 0)),
        ],
        out_specs=pl.BlockSpec((BLK, C), lambda i: (i, 0)),
        out_shape=jax.ShapeDtypeStruct((N, C), jnp.float32),
    )(p, zt1, dinv, cls_W, v2)


@jax.jit
def kernel(x, edge_index,
           c0_p0_W, c0_p0_b, c0_p1_W, c0_p1_b, c0_g0_W, c0_g0_b,
           c0_g1_W, c0_g1_b,
           c1_p0_W, c1_p0_b, c1_p1_W, c1_p1_b, c1_g0_W, c1_g0_b,
           c1_g1_W, c1_g1_b,
           cls_W, cls_b):
    src = edge_index[0]
    dst = edge_index[1]

    m0 = 2.0 * (c0_p0_W @ c0_g0_W) + c0_p1_W @ c0_g1_W
    v0 = 2.0 * (c0_p0_b @ c0_g0_W) + c0_p1_b @ c0_g1_W
    cb0 = 2.0 * c0_g0_b + c0_g1_b
    m1a = 2.0 * (c1_p0_W @ c1_g0_W)
    m1b = c1_p1_W @ c1_g1_W
    v1 = 2.0 * (c1_p0_b @ c1_g0_W) + c1_p1_b @ c1_g1_W + cb0 @ m1b
    cb1 = 2.0 * c1_g0_b + c1_g1_b
    v2 = cb1 @ cls_W + cls_b

    zeros1 = jnp.zeros((ROWS_A,), jnp.float32)
    zeros2 = jnp.zeros((ROWS_B, H), jnp.float32)

    src2 = src.reshape(NW * NCHUNK, K)
    dst2 = dst.reshape(NW * NCHUNK, K)
    deg_parts = _sc_degree(dst2, zeros1)
    dinv = lax.rsqrt(1.0 + deg_parts[:N] + deg_parts[N:]).reshape(N, 1)
    zt0, z1a = _tc_b(x, dinv, m0, m1a, v0.reshape(1, H))
    p0 = _sc_propagate(zt0, src2, dst2, zeros2)
    zt1 = _tc_c(p0, zt0, z1a, dinv, m1b, v1.reshape(1, H))
    p1 = _sc_propagate(zt1, src2, dst2, zeros2)
    return _tc_d(p1, zt1, dinv, cls_W, v2.reshape(1, C))
